# baseline (speedup 1.0000x reference)
import functools
import math

import jax
import jax.numpy as jnp
import numpy as np
from jax import lax
from jax.experimental import pallas as pl
from jax.experimental.pallas import tpu as pltpu
from jax.experimental.pallas import tpu_sc as plsc

VOCAB = 1000000
D = 64
SEQ = 200
SCALE = math.sqrt(D)

NC = 2
NS = 16
NW = NC * NS

CB = 200


def _pe_template(rows: int) -> np.ndarray:
    position = np.arange(SEQ, dtype=np.float32)[:, None]
    num_even = D // 2 + D % 2
    div_term = np.exp(
        np.arange(0, num_even, dtype=np.float32) * (-math.log(10000.0) / D)
    )
    pe = np.zeros((SEQ, D), dtype=np.float32)
    pe[:, 0::2] = np.sin(position * div_term[:num_even])
    pe[:, 1::2] = np.cos(position * div_term[: D // 2])
    return np.tile(pe, (rows // SEQ, 1)).astype(np.float32)


def _sc_embed(x_flat, table, pe_tile, n_rows):
    b_per_w = n_rows // NW
    nchunks = b_per_w // CB
    mesh = plsc.VectorSubcoreMesh(
        core_axis_name="c", subcore_axis_name="s", num_cores=NC, num_subcores=NS
    )

    @functools.partial(
        pl.kernel,
        out_type=jax.ShapeDtypeStruct((n_rows, D), jnp.float32),
        mesh=mesh,
        compiler_params=pltpu.CompilerParams(use_tc_tiling_on_sc=False),
        scratch_types=[
            pltpu.VMEM((b_per_w,), jnp.int32),
            pltpu.VMEM((2, CB, D), jnp.float32),
            pltpu.VMEM((2, CB, D), jnp.float32),
            pltpu.VMEM((CB, D), jnp.float32),
            pltpu.SemaphoreType.DMA,
            pltpu.SemaphoreType.DMA,
            pltpu.SemaphoreType.DMA,
            pltpu.SemaphoreType.DMA,
        ],
    )
    def k(x_hbm, table_hbm, pe_hbm, out_hbm, idx_v, gbuf, obuf, pe_v,
          sg0, sg1, sw0, sw1):
        sg = (sg0, sg1)
        sw = (sw0, sw1)
        wid = lax.axis_index("s") * NC + lax.axis_index("c")
        base = wid * b_per_w
        pltpu.sync_copy(pe_hbm, pe_v)
        pltpu.sync_copy(x_hbm.at[pl.ds(base, b_per_w)], idx_v)

        def start_gather(b, c):
            pltpu.async_copy(
                table_hbm.at[idx_v.at[pl.ds(c * CB, CB)]], gbuf.at[b], sg[b]
            )

        def drain_gather(b):
            pltpu.make_async_copy(table_hbm.at[idx_v.at[pl.ds(0, CB)]],
                                  gbuf.at[b], sg[b]).wait()

        def drain_write(b, c):
            pltpu.make_async_copy(obuf.at[b],
                                  out_hbm.at[pl.ds(base + c * CB, CB)],
                                  sw[b]).wait()

        start_gather(0, 0)
        start_gather(1, 1)

        @pl.loop(0, nchunks, step=2)
        def _outer(c0):
            for b in range(2):
                c = c0 + b
                drain_gather(b)

                @pl.when(c >= 2)
                def _():
                    drain_write(b, c - 2)

                @pl.loop(0, CB, unroll=4)
                def _row(r):
                    for j in range(D // 16):
                        s = pl.ds(j * 16, 16)
                        obuf[b, r, s] = gbuf[b, r, s] * SCALE + pe_v[r, s]

                pltpu.async_copy(
                    obuf.at[b], out_hbm.at[pl.ds(base + c * CB, CB)], sw[b]
                )

                @pl.when(c + 2 < nchunks)
                def _():
                    start_gather(b, c + 2)

        drain_write(0, nchunks - 2)
        drain_write(1, nchunks - 1)

    return k(x_flat, table, pe_tile)


def kernel(x, table):
    b, s = x.shape
    n_rows = b * s
    x_flat = x.reshape(n_rows).astype(jnp.int32)
    pe_tile = jnp.asarray(_pe_template(CB))
    out = _sc_embed(x_flat, table, pe_tile, n_rows)
    return out.reshape(b, s, D)
